# Initial kernel scaffold; baseline (speedup 1.0000x reference)
import jax
import jax.numpy as jnp
from jax.experimental import pallas as pl


def kernel(x, edge_index, edge_attr, batch, num_graphs, We1, be1, W1a, b1a, W1b, b1b, We2, be2, W2a, b2a, W2b, b2b, We3, be3, W3a, b3a, W3b, b3b, Wlin, blin, Wlin2, blin2):
    raise NotImplementedError("write your pallas kernel here")


# speedup vs baseline: 2.4860x; 2.4860x over previous
import functools

import jax
import jax.numpy as jnp
from jax import lax
from jax.experimental import pallas as pl
from jax.experimental.pallas import tpu as pltpu
from jax.experimental.pallas import tpu_sc as plsc

N = 10000
E = 320000
D = 128
DE = 16
G = 64
NCLS = 10

_NUM_WORKERS = 32
_EPW = E // _NUM_WORKERS
_C = 80
_NCHUNK = _EPW // _C
_NPAD = 10240
_RPW = _NPAD // 16
_ZB = 128


def _aggr_body(x_hbm, eb_hbm, src_hbm, dst_hbm, out_hbm,
               src_v, dst_v, xr_v, m_v, zb_v, acc_sh, sem):
    cid = lax.axis_index("c")
    sid = lax.axis_index("s")
    wid = sid * 2 + cid

    def _zrow(r, carry):
        for j in range(D // 16):
            zb_v[r, pl.ds(j * 16, 16)] = jnp.zeros((16,), jnp.float32)
        return carry

    lax.fori_loop(0, _ZB, _zrow, 0)
    row0 = sid * _RPW
    for k in range(_RPW // _ZB):
        pltpu.sync_copy(zb_v, acc_sh.at[pl.ds(row0 + k * _ZB, _ZB)])
    plsc.subcore_barrier()

    def _chunk(k, carry):
        base = wid * _EPW + k * _C
        pltpu.sync_copy(src_hbm.at[pl.ds(base, _C)], src_v)
        pltpu.sync_copy(dst_hbm.at[pl.ds(base, _C)], dst_v)
        pltpu.async_copy(x_hbm.at[src_v], xr_v, sem).wait()
        pltpu.sync_copy(eb_hbm.at[pl.ds(base, _C)], m_v)

        def _row(r, c2):
            for j in range(D // 16):
                s = pl.ds(j * 16, 16)
                m_v[r, s] = jnp.maximum(xr_v[r, s] + m_v[r, s], 0.0)
            return c2

        lax.fori_loop(0, _C, _row, 0)
        pltpu.sync_copy(m_v, acc_sh.at[dst_v], add=True)
        return carry

    lax.fori_loop(0, _NCHUNK, _chunk, 0)
    plsc.subcore_barrier()

    pltpu.sync_copy(acc_sh.at[pl.ds(row0, _RPW)],
                    out_hbm.at[cid, pl.ds(row0, _RPW)])


_aggr = pl.kernel(
    _aggr_body,
    out_type=jax.ShapeDtypeStruct((2, _NPAD, D), jnp.float32),
    mesh=plsc.VectorSubcoreMesh(core_axis_name="c", subcore_axis_name="s"),
    scratch_types=[
        pltpu.VMEM((_C,), jnp.int32),
        pltpu.VMEM((_C,), jnp.int32),
        pltpu.VMEM((_C, D), jnp.float32),
        pltpu.VMEM((_C, D), jnp.float32),
        pltpu.VMEM((_ZB, D), jnp.float32),
        pltpu.VMEM_SHARED((_NPAD, D), jnp.float32),
        pltpu.SemaphoreType.DMA,
    ],
)


_BE = 2000


def _edge_emb_body(ea_ref, w1_ref, b1_ref, w2_ref, b2_ref, w3_ref, b3_ref,
                   o1_ref, o2_ref, o3_ref):
    ea = ea_ref[...]
    o1_ref[...] = jnp.dot(ea, w1_ref[...],
                          preferred_element_type=jnp.float32) + b1_ref[...]
    o2_ref[...] = jnp.dot(ea, w2_ref[...],
                          preferred_element_type=jnp.float32) + b2_ref[...]
    o3_ref[...] = jnp.dot(ea, w3_ref[...],
                          preferred_element_type=jnp.float32) + b3_ref[...]


_edge_emb = pl.pallas_call(
    _edge_emb_body,
    grid=(E // _BE,),
    in_specs=[
        pl.BlockSpec((_BE, DE), lambda i: (i, 0)),
        pl.BlockSpec((DE, D), lambda i: (0, 0)),
        pl.BlockSpec((1, D), lambda i: (0, 0)),
        pl.BlockSpec((DE, D), lambda i: (0, 0)),
        pl.BlockSpec((1, D), lambda i: (0, 0)),
        pl.BlockSpec((DE, D), lambda i: (0, 0)),
        pl.BlockSpec((1, D), lambda i: (0, 0)),
    ],
    out_specs=[pl.BlockSpec((_BE, D), lambda i: (i, 0))] * 3,
    out_shape=[jax.ShapeDtypeStruct((E, D), jnp.float32)] * 3,
)


_BN = 1000


def _node_mlp_body(relu_out, x_ref, a0_ref, a1_ref, wa_ref, ba_ref,
                   wb_ref, bb_ref, o_ref):
    h = x_ref[...] + a0_ref[...] + a1_ref[...]
    t = jnp.maximum(
        jnp.dot(h, wa_ref[...], preferred_element_type=jnp.float32)
        + ba_ref[...], 0.0)
    y = jnp.dot(t, wb_ref[...],
                preferred_element_type=jnp.float32) + bb_ref[...]
    o_ref[...] = jnp.maximum(y, 0.0) if relu_out else y


def _make_node_mlp(relu_out):
    return pl.pallas_call(
        functools.partial(_node_mlp_body, relu_out),
        grid=(N // _BN,),
        in_specs=[
            pl.BlockSpec((_BN, D), lambda i: (i, 0)),
            pl.BlockSpec((_BN, D), lambda i: (i, 0)),
            pl.BlockSpec((_BN, D), lambda i: (i, 0)),
            pl.BlockSpec((D, D), lambda i: (0, 0)),
            pl.BlockSpec((1, D), lambda i: (0, 0)),
            pl.BlockSpec((D, D), lambda i: (0, 0)),
            pl.BlockSpec((1, D), lambda i: (0, 0)),
        ],
        out_specs=pl.BlockSpec((_BN, D), lambda i: (i, 0)),
        out_shape=jax.ShapeDtypeStruct((N, D), jnp.float32),
    )


_node_mlp_relu = _make_node_mlp(True)
_node_mlp_plain = _make_node_mlp(False)


_PB = 1000


def _pool_body(h_ref, b_ref, wl_ref, bl_ref, w2_ref, b2_ref, o_ref):
    sums = jnp.zeros((G, D), jnp.float32)
    cnt = jnp.zeros((G, 1), jnp.float32)
    for i in range(N // _PB):
        bb = b_ref[i, 0, :]
        onehot_t = (lax.broadcasted_iota(jnp.int32, (G, _PB), 0)
                    == bb[None, :]).astype(jnp.float32)
        hblk = h_ref[pl.ds(i * _PB, _PB), :]
        sums = sums + jnp.dot(onehot_t, hblk,
                              preferred_element_type=jnp.float32)
        cnt = cnt + jnp.sum(onehot_t, axis=1, keepdims=True)
    pooled = sums / jnp.maximum(cnt, 1.0)
    z = jnp.maximum(
        jnp.dot(pooled, wl_ref[...], preferred_element_type=jnp.float32)
        + bl_ref[...], 0.0)
    o_ref[...] = jnp.dot(z, w2_ref[...],
                         preferred_element_type=jnp.float32) + b2_ref[...]


_pool = pl.pallas_call(
    _pool_body,
    in_specs=[
        pl.BlockSpec((N, D), lambda: (0, 0)),
        pl.BlockSpec((N // _PB, 1, _PB), lambda: (0, 0, 0)),
        pl.BlockSpec((D, 256), lambda: (0, 0)),
        pl.BlockSpec((1, 256), lambda: (0, 0)),
        pl.BlockSpec((256, NCLS), lambda: (0, 0)),
        pl.BlockSpec((1, NCLS), lambda: (0, 0)),
    ],
    out_specs=pl.BlockSpec((G, NCLS), lambda: (0, 0)),
    out_shape=jax.ShapeDtypeStruct((G, NCLS), jnp.float32),
)


def kernel(x, edge_index, edge_attr, batch, num_graphs, We1, be1, W1a, b1a,
           W1b, b1b, We2, be2, W2a, b2a, W2b, b2b, We3, be3, W3a, b3a, W3b,
           b3b, Wlin, blin, Wlin2, blin2):
    src = edge_index[0]
    dst = edge_index[1]
    eb1, eb2, eb3 = _edge_emb(edge_attr, We1, be1.reshape(1, D),
                              We2, be2.reshape(1, D), We3, be3.reshape(1, D))
    a = _aggr(x, eb1, src, dst)
    h = _node_mlp_relu(x, a[0, :N], a[1, :N], W1a, b1a.reshape(1, D),
                       W1b, b1b.reshape(1, D))
    a = _aggr(h, eb2, src, dst)
    h = _node_mlp_relu(h, a[0, :N], a[1, :N], W2a, b2a.reshape(1, D),
                       W2b, b2b.reshape(1, D))
    a = _aggr(h, eb3, src, dst)
    h = _node_mlp_plain(h, a[0, :N], a[1, :N], W3a, b3a.reshape(1, D),
                        W3b, b3b.reshape(1, D))
    out = _pool(h, batch.reshape(N // _PB, 1, _PB),
                Wlin, blin.reshape(1, 256), Wlin2, blin2.reshape(1, NCLS))
    return out


# speedup vs baseline: 4.9632x; 1.9965x over previous
import functools

import jax
import jax.numpy as jnp
from jax import lax
from jax.experimental import pallas as pl
from jax.experimental.pallas import tpu as pltpu
from jax.experimental.pallas import tpu_sc as plsc

N = 10000
E = 320000
D = 128
DE = 16
G = 64
NCLS = 10

_NUM_WORKERS = 32
_EPW = E // _NUM_WORKERS
_C = 40
_NCHUNK = _EPW // _C
_NPAD = 10240
_RPW = _NPAD // 16


_NBUF = 4


def _aggr_body(x_hbm, eb_hbm, src_hbm, dst_hbm, out_hbm,
               src_v, dst_v, xr_v, m_v, acc_sh,
               isem, esem, gsem, scsem):
    cid = lax.axis_index("c")
    sid = lax.axis_index("s")
    wid = sid * 2 + cid

    def _fire_idx(c, b):
        base = wid * _EPW + c * _C
        pltpu.async_copy(src_hbm.at[pl.ds(base, _C)], src_v.at[b],
                         isem.at[b])
        pltpu.async_copy(dst_hbm.at[pl.ds(base, _C)], dst_v.at[b],
                         isem.at[b])

    def _wait_idx(b):
        pltpu.make_async_copy(src_hbm.at[pl.ds(0, _C)], src_v.at[b],
                              isem.at[b]).wait()
        pltpu.make_async_copy(dst_hbm.at[pl.ds(0, _C)], dst_v.at[b],
                              isem.at[b]).wait()

    def _fire_ebg(c, b):
        base = wid * _EPW + c * _C
        pltpu.async_copy(eb_hbm.at[pl.ds(base, _C)], m_v.at[b], esem.at[b])
        pltpu.async_copy(x_hbm.at[src_v.at[b]], xr_v.at[b], gsem.at[b])

    def _wait_ebg(b):
        pltpu.make_async_copy(eb_hbm.at[pl.ds(0, _C)], m_v.at[b],
                              esem.at[b]).wait()
        pltpu.make_async_copy(eb_hbm.at[pl.ds(0, _C)], xr_v.at[b],
                              gsem.at[b]).wait()

    def _drain_scatter(b):
        pltpu.make_async_copy(eb_hbm.at[pl.ds(0, _C)], m_v.at[b],
                              scsem.at[b]).wait()

    def _zrow(r, carry):
        for j in range(D // 16):
            m_v[0, r, pl.ds(j * 16, 16)] = jnp.zeros((16,), jnp.float32)
        return carry

    lax.fori_loop(0, _C, _zrow, 0)
    row0 = sid * _RPW
    for k in range(_RPW // _C):
        pltpu.sync_copy(m_v.at[0], acc_sh.at[pl.ds(row0 + k * _C, _C)])
    plsc.subcore_barrier()

    _fire_idx(0, 0)
    _fire_idx(1, 1)
    _fire_idx(2, 2)
    _wait_idx(0)
    _fire_ebg(0, 0)
    _wait_idx(1)
    _fire_ebg(1, 1)

    def _process(c, b):
        _wait_ebg(b)

        def _row(r, carry):
            for j in range(D // 16):
                s = pl.ds(j * 16, 16)
                m_v[b, r, s] = jnp.maximum(xr_v[b, r, s] + m_v[b, r, s], 0.0)
            return carry

        lax.fori_loop(0, _C, _row, 0)
        pltpu.async_copy(m_v.at[b], acc_sh.at[dst_v.at[b]], scsem.at[b],
                         add=True)

        @pl.when(c + 3 < _NCHUNK)
        def _():
            _fire_idx(c + 3, (b + 3) % _NBUF)

        @pl.when(c >= 2)
        def _():
            _drain_scatter((b + 2) % _NBUF)

        @pl.when(c + 2 < _NCHUNK)
        def _():
            _wait_idx((b + 2) % _NBUF)
            _fire_ebg(c + 2, (b + 2) % _NBUF)

    def _outer(t, carry):
        for i in range(_NBUF):
            _process(t * _NBUF + i, i)
        return carry

    _MAIN = (_NCHUNK // _NBUF) * _NBUF
    lax.fori_loop(0, _MAIN // _NBUF, _outer, 0)
    for c in range(_MAIN, _NCHUNK):
        _process(c, c % _NBUF)
    for c in (_NCHUNK - 2, _NCHUNK - 1):
        _drain_scatter(c % _NBUF)
    plsc.subcore_barrier()

    pltpu.sync_copy(acc_sh.at[pl.ds(row0, _RPW)],
                    out_hbm.at[cid, pl.ds(row0, _RPW)])


_aggr = pl.kernel(
    _aggr_body,
    out_type=jax.ShapeDtypeStruct((2, _NPAD, D), jnp.float32),
    mesh=plsc.VectorSubcoreMesh(core_axis_name="c", subcore_axis_name="s"),
    scratch_types=[
        pltpu.VMEM((_NBUF, _C), jnp.int32),
        pltpu.VMEM((_NBUF, _C), jnp.int32),
        pltpu.VMEM((_NBUF, _C, D), jnp.float32),
        pltpu.VMEM((_NBUF, _C, D), jnp.float32),
        pltpu.VMEM_SHARED((_NPAD, D), jnp.float32),
        pltpu.SemaphoreType.DMA((_NBUF,)),
        pltpu.SemaphoreType.DMA((_NBUF,)),
        pltpu.SemaphoreType.DMA((_NBUF,)),
        pltpu.SemaphoreType.DMA((_NBUF,)),
    ],
)


_BE = 2000


def _edge_emb_body(ea_ref, w1_ref, b1_ref, w2_ref, b2_ref, w3_ref, b3_ref,
                   o1_ref, o2_ref, o3_ref):
    ea = ea_ref[...]
    o1_ref[...] = jnp.dot(ea, w1_ref[...],
                          preferred_element_type=jnp.float32) + b1_ref[...]
    o2_ref[...] = jnp.dot(ea, w2_ref[...],
                          preferred_element_type=jnp.float32) + b2_ref[...]
    o3_ref[...] = jnp.dot(ea, w3_ref[...],
                          preferred_element_type=jnp.float32) + b3_ref[...]


_edge_emb = pl.pallas_call(
    _edge_emb_body,
    grid=(E // _BE,),
    in_specs=[
        pl.BlockSpec((_BE, DE), lambda i: (i, 0)),
        pl.BlockSpec((DE, D), lambda i: (0, 0)),
        pl.BlockSpec((1, D), lambda i: (0, 0)),
        pl.BlockSpec((DE, D), lambda i: (0, 0)),
        pl.BlockSpec((1, D), lambda i: (0, 0)),
        pl.BlockSpec((DE, D), lambda i: (0, 0)),
        pl.BlockSpec((1, D), lambda i: (0, 0)),
    ],
    out_specs=[pl.BlockSpec((_BE, D), lambda i: (i, 0))] * 3,
    out_shape=[jax.ShapeDtypeStruct((E, D), jnp.float32)] * 3,
)


_BN = 1000


def _node_mlp_body(relu_out, x_ref, a0_ref, a1_ref, wa_ref, ba_ref,
                   wb_ref, bb_ref, o_ref):
    h = x_ref[...] + a0_ref[...] + a1_ref[...]
    t = jnp.maximum(
        jnp.dot(h, wa_ref[...], preferred_element_type=jnp.float32)
        + ba_ref[...], 0.0)
    y = jnp.dot(t, wb_ref[...],
                preferred_element_type=jnp.float32) + bb_ref[...]
    o_ref[...] = jnp.maximum(y, 0.0) if relu_out else y


def _make_node_mlp(relu_out):
    return pl.pallas_call(
        functools.partial(_node_mlp_body, relu_out),
        grid=(N // _BN,),
        in_specs=[
            pl.BlockSpec((_BN, D), lambda i: (i, 0)),
            pl.BlockSpec((_BN, D), lambda i: (i, 0)),
            pl.BlockSpec((_BN, D), lambda i: (i, 0)),
            pl.BlockSpec((D, D), lambda i: (0, 0)),
            pl.BlockSpec((1, D), lambda i: (0, 0)),
            pl.BlockSpec((D, D), lambda i: (0, 0)),
            pl.BlockSpec((1, D), lambda i: (0, 0)),
        ],
        out_specs=pl.BlockSpec((_BN, D), lambda i: (i, 0)),
        out_shape=jax.ShapeDtypeStruct((N, D), jnp.float32),
    )


_node_mlp_relu = _make_node_mlp(True)
_node_mlp_plain = _make_node_mlp(False)


_PB = 1000


def _pool_body(h_ref, b_ref, wl_ref, bl_ref, w2_ref, b2_ref, o_ref):
    sums = jnp.zeros((G, D), jnp.float32)
    cnt = jnp.zeros((G, 1), jnp.float32)
    for i in range(N // _PB):
        bb = b_ref[i, 0, :]
        onehot_t = (lax.broadcasted_iota(jnp.int32, (G, _PB), 0)
                    == bb[None, :]).astype(jnp.float32)
        hblk = h_ref[pl.ds(i * _PB, _PB), :]
        sums = sums + jnp.dot(onehot_t, hblk,
                              preferred_element_type=jnp.float32)
        cnt = cnt + jnp.sum(onehot_t, axis=1, keepdims=True)
    pooled = sums / jnp.maximum(cnt, 1.0)
    z = jnp.maximum(
        jnp.dot(pooled, wl_ref[...], preferred_element_type=jnp.float32)
        + bl_ref[...], 0.0)
    o_ref[...] = jnp.dot(z, w2_ref[...],
                         preferred_element_type=jnp.float32) + b2_ref[...]


_pool = pl.pallas_call(
    _pool_body,
    in_specs=[
        pl.BlockSpec((N, D), lambda: (0, 0)),
        pl.BlockSpec((N // _PB, 1, _PB), lambda: (0, 0, 0)),
        pl.BlockSpec((D, 256), lambda: (0, 0)),
        pl.BlockSpec((1, 256), lambda: (0, 0)),
        pl.BlockSpec((256, NCLS), lambda: (0, 0)),
        pl.BlockSpec((1, NCLS), lambda: (0, 0)),
    ],
    out_specs=pl.BlockSpec((G, NCLS), lambda: (0, 0)),
    out_shape=jax.ShapeDtypeStruct((G, NCLS), jnp.float32),
)


def kernel(x, edge_index, edge_attr, batch, num_graphs, We1, be1, W1a, b1a,
           W1b, b1b, We2, be2, W2a, b2a, W2b, b2b, We3, be3, W3a, b3a, W3b,
           b3b, Wlin, blin, Wlin2, blin2):
    src = edge_index[0]
    dst = edge_index[1]
    eb1, eb2, eb3 = _edge_emb(edge_attr, We1, be1.reshape(1, D),
                              We2, be2.reshape(1, D), We3, be3.reshape(1, D))
    a = _aggr(x, eb1, src, dst)
    h = _node_mlp_relu(x, a[0, :N], a[1, :N], W1a, b1a.reshape(1, D),
                       W1b, b1b.reshape(1, D))
    a = _aggr(h, eb2, src, dst)
    h = _node_mlp_relu(h, a[0, :N], a[1, :N], W2a, b2a.reshape(1, D),
                       W2b, b2b.reshape(1, D))
    a = _aggr(h, eb3, src, dst)
    h = _node_mlp_plain(h, a[0, :N], a[1, :N], W3a, b3a.reshape(1, D),
                        W3b, b3b.reshape(1, D))
    out = _pool(h, batch.reshape(N // _PB, 1, _PB),
                Wlin, blin.reshape(1, 256), Wlin2, blin2.reshape(1, NCLS))
    return out
